# baseline (speedup 1.0000x reference)
import functools

import jax
import jax.numpy as jnp
from jax import lax
from jax.experimental import pallas as pl
from jax.experimental.pallas import tpu as pltpu
from jax.experimental.pallas import tpu_sc as plsc

S, B, D = 4096, 2, 1024
L = 16
NCORES, NSUB = 2, 16
NW = NCORES * NSUB
P = 8
ROWS = P * B
POS_PER_W = S // NW
CHUNKS = POS_PER_W // P

_mesh = plsc.VectorSubcoreMesh(core_axis_name="c", subcore_axis_name="s")


@functools.partial(
    pl.kernel,
    out_type=jax.ShapeDtypeStruct((S * B, D), jnp.float32),
    mesh=_mesh,
    scratch_types=[
        pltpu.VMEM((2, ROWS, D), jnp.float32),
        pltpu.VMEM((2, P, D), jnp.float32),
        pltpu.VMEM((2, ROWS, D), jnp.float32),
        pltpu.SemaphoreType.DMA((2,)),
        pltpu.SemaphoreType.DMA((2,)),
        pltpu.SemaphoreType.DMA((2,)),
    ],
)
def _sc_add(x_hbm, pe_hbm, out_hbm, xb, peb, ob, sx, sp, so):
    wid = lax.axis_index("s") * NCORES + lax.axis_index("c")
    pos0 = wid * POS_PER_W
    row0 = pos0 * B

    def start_in(g, slot):
        cx = pltpu.async_copy(
            x_hbm.at[pl.ds(row0 + g * ROWS, ROWS)], xb.at[slot], sx.at[slot])
        cp = pltpu.async_copy(
            pe_hbm.at[pl.ds(pos0 + g * P, P)], peb.at[slot], sp.at[slot])
        return cx, cp

    def compute(slot):
        def body(p, carry):
            for j in range(D // L):
                dsl = pl.ds(j * L, L)
                pv = peb[slot, p, dsl]
                ob[slot, 2 * p, dsl] = xb[slot, 2 * p, dsl] + pv
                ob[slot, 2 * p + 1, dsl] = xb[slot, 2 * p + 1, dsl] + pv
            return carry
        lax.fori_loop(0, P, body, 0)

    pending_in = {}
    pending_out = {}
    pending_in[0] = start_in(0, 0)
    for g in range(CHUNKS):
        slot = g % 2
        if g + 1 < CHUNKS:
            pending_in[g + 1] = start_in(g + 1, (g + 1) % 2)
        cx, cp = pending_in.pop(g)
        cx.wait()
        cp.wait()
        if g >= 2:
            pending_out.pop(g - 2).wait()
        compute(slot)
        pending_out[g] = pltpu.async_copy(
            ob.at[slot], out_hbm.at[pl.ds(row0 + g * ROWS, ROWS)], so.at[slot])
    for g in sorted(pending_out):
        pending_out.pop(g).wait()


def kernel(x, pe):
    s, b, d = x.shape
    out = _sc_add(x.reshape(s * b, d), pe[:s])
    return out.reshape(s, b, d)


# speedup vs baseline: 1.9614x; 1.9614x over previous
import functools

import jax
import jax.numpy as jnp
from jax import lax
from jax.experimental import pallas as pl
from jax.experimental.pallas import tpu as pltpu
from jax.experimental.pallas import tpu_sc as plsc

S, B, D = 4096, 2, 1024
L = 16
NCORES, NSUB = 2, 16
NW = NCORES * NSUB
P = 8
POS_PER_W = S // NW
CHUNKS = POS_PER_W // P
NBUF = 3

_mesh = plsc.VectorSubcoreMesh(core_axis_name="c", subcore_axis_name="s")


@functools.partial(
    pl.kernel,
    out_type=jax.ShapeDtypeStruct((S, B, D), jnp.float32),
    mesh=_mesh,
    scratch_types=[
        pltpu.VMEM((NBUF, P, B, D), jnp.float32),
        pltpu.VMEM((NBUF, P, D), jnp.float32),
        pltpu.SemaphoreType.DMA((NBUF,)),
        pltpu.SemaphoreType.DMA((NBUF,)),
        pltpu.SemaphoreType.DMA((NBUF,)),
    ],
)
def _sc_add(x_hbm, pe_hbm, out_hbm, xb, peb, sx, sp, so):
    wid = lax.axis_index("s") * NCORES + lax.axis_index("c")
    pos0 = wid * POS_PER_W

    def start_in(g, slot):
        sl = pl.ds(pos0 + g * P, P)
        return (pltpu.async_copy(x_hbm.at[sl], xb.at[slot], sx.at[slot]),
                pltpu.async_copy(pe_hbm.at[sl], peb.at[slot], sp.at[slot]))

    def compute(slot):
        def body(p, carry):
            for j in range(D // L):
                dsl = pl.ds(j * L, L)
                pv = peb[slot, p, dsl]
                xb[slot, p, 0, dsl] += pv
                xb[slot, p, 1, dsl] += pv
            return carry
        lax.fori_loop(0, P, body, 0)

    in_h, out_h = {}, {}
    in_h[0] = start_in(0, 0)
    if CHUNKS > 1:
        in_h[1] = start_in(1, 1)
    for g in range(CHUNKS):
        slot = g % NBUF
        if g + 2 < CHUNKS:
            if g - 1 >= 0:
                out_h.pop(g - 1).wait()
            in_h[g + 2] = start_in(g + 2, (g + 2) % NBUF)
        cx, cp = in_h.pop(g)
        cx.wait()
        cp.wait()
        compute(slot)
        out_h[g] = pltpu.async_copy(
            xb.at[slot], out_hbm.at[pl.ds(pos0 + g * P, P)], so.at[slot])
    for g in sorted(out_h):
        out_h.pop(g).wait()


def kernel(x, pe):
    return _sc_add(x, pe)


# speedup vs baseline: 2.9660x; 1.5121x over previous
import functools

import jax
import jax.numpy as jnp
from jax import lax
from jax.experimental import pallas as pl
from jax.experimental.pallas import tpu as pltpu
from jax.experimental.pallas import tpu_sc as plsc

S, B, D = 4096, 2, 1024
L = 16
NCORES, NSUB = 2, 16
NW = NCORES * NSUB
P = 8
POS_PER_W = S // NW
CHUNKS = POS_PER_W // P
NBUF = 3

_mesh = plsc.VectorSubcoreMesh(core_axis_name="c", subcore_axis_name="s")


@functools.partial(
    pl.kernel,
    out_type=jax.ShapeDtypeStruct((S, B, D), jnp.float32),
    mesh=_mesh,
    scratch_types=[
        pltpu.VMEM((NBUF, P, B, D), jnp.float32),
        pltpu.VMEM((NBUF, P, D), jnp.float32),
        pltpu.SemaphoreType.DMA((NBUF,)),
        pltpu.SemaphoreType.DMA((NBUF,)),
        pltpu.SemaphoreType.DMA((NBUF,)),
    ],
)
def _sc_add(x_hbm, pe_hbm, out_hbm, xb, peb, sx, sp, so):
    wid = lax.axis_index("s") * NCORES + lax.axis_index("c")
    pos0 = wid * POS_PER_W

    def start_in(g, slot):
        sl = pl.ds(pos0 + g * P, P)
        return (pltpu.async_copy(x_hbm.at[sl], xb.at[slot], sx.at[slot]),
                pltpu.async_copy(pe_hbm.at[sl], peb.at[slot], sp.at[slot]))

    G = 8

    def compute(slot):
        def body(p, carry):
            for j0 in range(0, D // L, G):
                sls = [pl.ds((j0 + j) * L, L) for j in range(G)]
                pvs = [peb[slot, p, dsl] for dsl in sls]
                a0 = [xb[slot, p, 0, dsl] + pv for dsl, pv in zip(sls, pvs)]
                a1 = [xb[slot, p, 1, dsl] + pv for dsl, pv in zip(sls, pvs)]
                for dsl, v in zip(sls, a0):
                    xb[slot, p, 0, dsl] = v
                for dsl, v in zip(sls, a1):
                    xb[slot, p, 1, dsl] = v
            return carry
        lax.fori_loop(0, P, body, 0)

    in_h, out_h = {}, {}
    in_h[0] = start_in(0, 0)
    if CHUNKS > 1:
        in_h[1] = start_in(1, 1)
    for g in range(CHUNKS):
        slot = g % NBUF
        if g + 2 < CHUNKS:
            if g - 1 >= 0:
                out_h.pop(g - 1).wait()
            in_h[g + 2] = start_in(g + 2, (g + 2) % NBUF)
        cx, cp = in_h.pop(g)
        cx.wait()
        cp.wait()
        compute(slot)
        out_h[g] = pltpu.async_copy(
            xb.at[slot], out_hbm.at[pl.ds(pos0 + g * P, P)], so.at[slot])
    for g in sorted(out_h):
        out_h.pop(g).wait()


def kernel(x, pe):
    return _sc_add(x, pe)


# speedup vs baseline: 2.9969x; 1.0104x over previous
import functools

import jax
import jax.numpy as jnp
from jax import lax
from jax.experimental import pallas as pl
from jax.experimental.pallas import tpu as pltpu
from jax.experimental.pallas import tpu_sc as plsc

S, B, D = 4096, 2, 1024
L = 16
NCORES, NSUB = 2, 16
NW = NCORES * NSUB
P = 8
POS_PER_W = S // NW
CHUNKS = POS_PER_W // P
NBUF = 4

_mesh = plsc.VectorSubcoreMesh(core_axis_name="c", subcore_axis_name="s")


@functools.partial(
    pl.kernel,
    out_type=jax.ShapeDtypeStruct((S, B, D), jnp.float32),
    mesh=_mesh,
    scratch_types=[
        pltpu.VMEM((NBUF, P, B, D), jnp.float32),
        pltpu.VMEM((NBUF, P, D), jnp.float32),
        pltpu.SemaphoreType.DMA((NBUF,)),
        pltpu.SemaphoreType.DMA((NBUF,)),
        pltpu.SemaphoreType.DMA((NBUF,)),
    ],
)
def _sc_add(x_hbm, pe_hbm, out_hbm, xb, peb, sx, sp, so):
    wid = lax.axis_index("s") * NCORES + lax.axis_index("c")
    pos0 = wid * POS_PER_W

    def start_in(g, slot):
        sl = pl.ds(pos0 + g * P, P)
        return (pltpu.async_copy(x_hbm.at[sl], xb.at[slot], sx.at[slot]),
                pltpu.async_copy(pe_hbm.at[sl], peb.at[slot], sp.at[slot]))

    G = 8

    def compute(slot):
        def body(p, carry):
            for j0 in range(0, D // L, G):
                sls = [pl.ds((j0 + j) * L, L) for j in range(G)]
                pvs = [peb[slot, p, dsl] for dsl in sls]
                a0 = [xb[slot, p, 0, dsl] + pv for dsl, pv in zip(sls, pvs)]
                a1 = [xb[slot, p, 1, dsl] + pv for dsl, pv in zip(sls, pvs)]
                for dsl, v in zip(sls, a0):
                    xb[slot, p, 0, dsl] = v
                for dsl, v in zip(sls, a1):
                    xb[slot, p, 1, dsl] = v
            return carry
        lax.fori_loop(0, P, body, 0)

    in_h, out_h = {}, {}
    for g0 in range(min(NBUF - 1, CHUNKS)):
        in_h[g0] = start_in(g0, g0)
    for g in range(CHUNKS):
        slot = g % NBUF
        if g + 3 < CHUNKS:
            if g - 1 >= 0:
                out_h.pop(g - 1).wait()
            in_h[g + 3] = start_in(g + 3, (g + 3) % NBUF)
        cx, cp = in_h.pop(g)
        cx.wait()
        cp.wait()
        compute(slot)
        out_h[g] = pltpu.async_copy(
            xb.at[slot], out_hbm.at[pl.ds(pos0 + g * P, P)], so.at[slot])
    for g in sorted(out_h):
        out_h.pop(g).wait()


def kernel(x, pe):
    return _sc_add(x, pe)


# speedup vs baseline: 3.4716x; 1.1584x over previous
import functools

import jax
import jax.numpy as jnp
from jax import lax
from jax.experimental import pallas as pl
from jax.experimental.pallas import tpu as pltpu
from jax.experimental.pallas import tpu_sc as plsc

S, B, D = 4096, 2, 1024
L = 16
NCORES, NSUB = 2, 16
NW = NCORES * NSUB
P = 8
POS_PER_W = S // NW
CHUNKS = POS_PER_W // P
NBUF = 4
G = 8

_mesh = plsc.VectorSubcoreMesh(core_axis_name="c", subcore_axis_name="s")


@functools.partial(
    pl.kernel,
    out_type=jax.ShapeDtypeStruct((S, B, D), jnp.float32),
    mesh=_mesh,
    scratch_types=[
        pltpu.VMEM((NBUF, P, B, D), jnp.float32),
        pltpu.VMEM((NBUF, P, D), jnp.float32),
        pltpu.SemaphoreType.DMA((NBUF,)),
        pltpu.SemaphoreType.DMA((NBUF,)),
        pltpu.SemaphoreType.DMA((NBUF,)),
    ],
)
def _sc_add(x_hbm, pe_hbm, out_hbm, xb, peb, sx, sp, so):
    wid = lax.axis_index("s") * NCORES + lax.axis_index("c")
    pos0 = wid * POS_PER_W

    def in_copies(g, slot):
        sl = pl.ds(pos0 + g * P, P)
        return (pltpu.make_async_copy(x_hbm.at[sl], xb.at[slot], sx.at[slot]),
                pltpu.make_async_copy(pe_hbm.at[sl], peb.at[slot], sp.at[slot]))

    def out_copy(g, slot):
        return pltpu.make_async_copy(
            xb.at[slot], out_hbm.at[pl.ds(pos0 + g * P, P)], so.at[slot])

    def compute(slot):
        def body(p, carry):
            for j0 in range(0, D // L, G):
                sls = [pl.ds((j0 + j) * L, L) for j in range(G)]
                pvs = [peb[slot, p, dsl] for dsl in sls]
                a0 = [xb[slot, p, 0, dsl] + pv for dsl, pv in zip(sls, pvs)]
                a1 = [xb[slot, p, 1, dsl] + pv for dsl, pv in zip(sls, pvs)]
                for dsl, v in zip(sls, a0):
                    xb[slot, p, 0, dsl] = v
                for dsl, v in zip(sls, a1):
                    xb[slot, p, 1, dsl] = v
            return carry
        lax.fori_loop(0, P, body, 0)

    for g0 in range(NBUF - 1):
        cx, cp = in_copies(g0, g0)
        cx.start()
        cp.start()

    def chunk_body(g, carry):
        slot = lax.rem(g, NBUF)

        @pl.when(g + NBUF - 1 < CHUNKS)
        def _():
            @pl.when(g >= 1)
            def _():
                out_copy(g - 1, lax.rem(g - 1, NBUF)).wait()
            nslot = lax.rem(g + NBUF - 1, NBUF)
            cx, cp = in_copies(g + NBUF - 1, nslot)
            cx.start()
            cp.start()

        cx, cp = in_copies(g, slot)
        cx.wait()
        cp.wait()
        compute(slot)
        out_copy(g, slot).start()
        return carry

    lax.fori_loop(0, CHUNKS, chunk_body, 0)
    for g in range(CHUNKS - NBUF, CHUNKS):
        out_copy(g, g % NBUF).wait()


def kernel(x, pe):
    return _sc_add(x, pe)


# speedup vs baseline: 3.5231x; 1.0148x over previous
import functools

import jax
import jax.numpy as jnp
from jax import lax
from jax.experimental import pallas as pl
from jax.experimental.pallas import tpu as pltpu
from jax.experimental.pallas import tpu_sc as plsc

S, B, D = 4096, 2, 1024
L = 16
NCORES, NSUB = 2, 16
NW = NCORES * NSUB
P = 8
POS_PER_W = S // NW
CHUNKS = POS_PER_W // P
NBUF = 5
G = 8

_mesh = plsc.VectorSubcoreMesh(core_axis_name="c", subcore_axis_name="s")


@functools.partial(
    pl.kernel,
    out_type=jax.ShapeDtypeStruct((S, B, D), jnp.float32),
    mesh=_mesh,
    scratch_types=[
        pltpu.VMEM((NBUF, P, B, D), jnp.float32),
        pltpu.VMEM((NBUF, P, D), jnp.float32),
        pltpu.SemaphoreType.DMA((NBUF,)),
        pltpu.SemaphoreType.DMA((NBUF,)),
        pltpu.SemaphoreType.DMA((NBUF,)),
    ],
)
def _sc_add(x_hbm, pe_hbm, out_hbm, xb, peb, sx, sp, so):
    wid = lax.axis_index("s") * NCORES + lax.axis_index("c")
    pos0 = wid * POS_PER_W

    def in_copies(g, slot):
        sl = pl.ds(pos0 + g * P, P)
        return (pltpu.make_async_copy(x_hbm.at[sl], xb.at[slot], sx.at[slot]),
                pltpu.make_async_copy(pe_hbm.at[sl], peb.at[slot], sp.at[slot]))

    def out_copy(g, slot):
        return pltpu.make_async_copy(
            xb.at[slot], out_hbm.at[pl.ds(pos0 + g * P, P)], so.at[slot])

    def compute(slot):
        def body(p, carry):
            for j0 in range(0, D // L, G):
                sls = [pl.ds((j0 + j) * L, L) for j in range(G)]
                pvs = [peb[slot, p, dsl] for dsl in sls]
                a0 = [xb[slot, p, 0, dsl] + pv for dsl, pv in zip(sls, pvs)]
                a1 = [xb[slot, p, 1, dsl] + pv for dsl, pv in zip(sls, pvs)]
                for dsl, v in zip(sls, a0):
                    xb[slot, p, 0, dsl] = v
                for dsl, v in zip(sls, a1):
                    xb[slot, p, 1, dsl] = v
            return carry
        lax.fori_loop(0, P, body, 0)

    for g0 in range(NBUF - 1):
        cx, cp = in_copies(g0, g0)
        cx.start()
        cp.start()

    def chunk_body(g, carry):
        slot = lax.rem(g, NBUF)

        @pl.when(g + NBUF - 1 < CHUNKS)
        def _():
            @pl.when(g >= 1)
            def _():
                out_copy(g - 1, lax.rem(g - 1, NBUF)).wait()
            nslot = lax.rem(g + NBUF - 1, NBUF)
            cx, cp = in_copies(g + NBUF - 1, nslot)
            cx.start()
            cp.start()

        cx, cp = in_copies(g, slot)
        cx.wait()
        cp.wait()
        compute(slot)
        out_copy(g, slot).start()
        return carry

    lax.fori_loop(0, CHUNKS, chunk_body, 0)
    for g in range(CHUNKS - NBUF, CHUNKS):
        out_copy(g, g % NBUF).wait()


def kernel(x, pe):
    return _sc_add(x, pe)
